# baseline (speedup 1.0000x reference)
import jax
import jax.numpy as jnp
from jax.experimental import pallas as pl
from jax.experimental.pallas import tpu as pltpu

N = 10000
F = 128
BLK = 1024
NBLK = (N + BLK - 1) // BLK
NPAD = NBLK * BLK
TAIL = N - (NBLK - 1) * BLK
TOT = NBLK + NBLK * (NBLK - 1)


def _ik(t):
    u = t - NBLK
    i = jnp.where(t < NBLK, t, u // (NBLK - 1))
    kk = u % (NBLK - 1)
    k_off = jnp.where(kk >= i, kk + 1, kk)
    k = jnp.where(t < NBLK, t, k_off)
    return i, k


def _kern(adj_ref, x_ref, w_ref, b_ref, out_ref, y_ref, s16_ref, rx_ref, rb_ref):
    t = pl.program_id(0)
    i, k = _ik(t)
    a = adj_ref[...]

    @pl.when(t < NBLK)
    def _diag():
        ri = jax.lax.broadcasted_iota(jnp.int32, (BLK, BLK), 0)
        ci = jax.lax.broadcasted_iota(jnp.int32, (BLK, BLK), 1)
        col_ok = t * BLK + ci < N
        row_ok = t * BLK + ri < N

        d = jnp.sum(jnp.where((ri == ci) & col_ok, a, 0.0), axis=0, keepdims=True)

        xr = jax.lax.broadcasted_iota(jnp.int32, (BLK, F), 0)
        xm = jnp.where(t * BLK + xr < N, x_ref[...], 0.0)
        s16 = jnp.dot(xm, w_ref[...], preferred_element_type=jnp.float32).astype(
            jnp.bfloat16
        )
        s16_ref[pl.ds(t * BLK, BLK)] = s16

        @pl.when(t == 0)
        def _():
            rx_ref[...] = jnp.zeros_like(rx_ref)

        rx_ref[...] += jnp.dot(d, xm, preferred_element_type=jnp.float32)

        a16 = jnp.where(col_ok & row_ok, a, 0.0).astype(jnp.bfloat16)
        y_ref[pl.ds(t * BLK, BLK)] = jnp.dot(
            a16, s16, preferred_element_type=jnp.float32
        )

        @pl.when(t == NBLK - 1)
        def _():
            rb_ref[...] = (
                jnp.dot(rx_ref[...], w_ref[...], preferred_element_type=jnp.float32)
                + b_ref[...].reshape(1, F)
            )

    @pl.when(t >= NBLK)
    def _offdiag():
        s16k = s16_ref[pl.ds(k * BLK, BLK)]

        @pl.when(k == NBLK - 1)
        def _():
            ci = jax.lax.broadcasted_iota(jnp.int32, (BLK, BLK), 1)
            a16 = jnp.where(ci < TAIL, a, 0.0).astype(jnp.bfloat16)
            y_ref[pl.ds(i * BLK, BLK)] += jnp.dot(
                a16, s16k, preferred_element_type=jnp.float32
            )

        @pl.when(k != NBLK - 1)
        def _():
            y_ref[pl.ds(i * BLK, BLK)] += jnp.dot(
                a.astype(jnp.bfloat16), s16k, preferred_element_type=jnp.float32
            )

    @pl.when(t == TOT - 1)
    def _finish():
        out_ref[...] = y_ref[pl.ds(0, N)] + rb_ref[...]


@jax.jit
def kernel(input, adj, W, b):
    return pl.pallas_call(
        _kern,
        grid=(TOT,),
        in_specs=[
            pl.BlockSpec((BLK, BLK), lambda t: _ik(t)),
            pl.BlockSpec((BLK, F), lambda t: (jnp.where(t < NBLK, t, 0), 0)),
            pl.BlockSpec((F, F), lambda t: (0, 0)),
            pl.BlockSpec((F,), lambda t: (0,)),
        ],
        out_specs=pl.BlockSpec((N, F), lambda t: (0, 0)),
        out_shape=jax.ShapeDtypeStruct((N, F), jnp.float32),
        scratch_shapes=[
            pltpu.VMEM((NPAD, F), jnp.float32),
            pltpu.VMEM((NPAD, F), jnp.bfloat16),
            pltpu.VMEM((1, F), jnp.float32),
            pltpu.VMEM((1, F), jnp.float32),
        ],
    )(adj, input, W, b)


# speedup vs baseline: 1.0920x; 1.0920x over previous
import jax
import jax.numpy as jnp
from jax.experimental import pallas as pl
from jax.experimental.pallas import tpu as pltpu

N = 10000
F = 128
BM = 200
NB = N // BM


def _kern(adj_ref, x_ref, w_ref, b_ref, out_ref, s16_ref, d_ref):
    i = pl.program_id(0)
    a = adj_ref[...]

    @pl.when(i == 0)
    def _():
        s16_ref[...] = jnp.dot(
            x_ref[...], w_ref[...], preferred_element_type=jnp.float32
        ).astype(jnp.bfloat16)
        d_ref[...] = jnp.zeros_like(d_ref)

    ri = jax.lax.broadcasted_iota(jnp.int32, (BM, N), 0)
    ci = jax.lax.broadcasted_iota(jnp.int32, (BM, N), 1)
    d_ref[...] += jnp.sum(
        jnp.where(ci == ri + i * BM, a, 0.0), axis=0, keepdims=True
    )

    out_ref[pl.ds(i * BM, BM)] = jnp.dot(
        a.astype(jnp.bfloat16), s16_ref[...], preferred_element_type=jnp.float32
    )

    @pl.when(i == NB - 1)
    def _():
        rb = (
            jnp.dot(
                d_ref[...].astype(jnp.bfloat16),
                s16_ref[...],
                preferred_element_type=jnp.float32,
            )
            + b_ref[...].reshape(1, F)
        )
        out_ref[...] += rb


@jax.jit
def kernel(input, adj, W, b):
    return pl.pallas_call(
        _kern,
        grid=(NB,),
        in_specs=[
            pl.BlockSpec((BM, N), lambda i: (i, 0)),
            pl.BlockSpec((N, F), lambda i: (0, 0)),
            pl.BlockSpec((F, F), lambda i: (0, 0)),
            pl.BlockSpec((F,), lambda i: (0,)),
        ],
        out_specs=pl.BlockSpec((N, F), lambda i: (0, 0)),
        out_shape=jax.ShapeDtypeStruct((N, F), jnp.float32),
        scratch_shapes=[
            pltpu.VMEM((N, F), jnp.bfloat16),
            pltpu.VMEM((1, N), jnp.float32),
        ],
    )(adj, input, W, b)


# speedup vs baseline: 1.1417x; 1.0455x over previous
import jax
import jax.numpy as jnp
from jax.experimental import pallas as pl
from jax.experimental.pallas import tpu as pltpu

N = 10000
F = 128
BM = 400
NB = N // BM


def _prep_kernel(x_ref, w_ref, s16_ref):
    s16_ref[...] = jnp.dot(
        x_ref[...].astype(jnp.bfloat16),
        w_ref[...].astype(jnp.bfloat16),
        preferred_element_type=jnp.float32,
    ).astype(jnp.bfloat16)


def _main_kernel(adj_ref, s16_ref, b_ref, out_ref, d_ref):
    i = pl.program_id(0)
    a = adj_ref[...]

    @pl.when(i == 0)
    def _():
        d_ref[...] = jnp.zeros_like(d_ref)

    ri = jax.lax.broadcasted_iota(jnp.int32, (BM, N), 0)
    ci = jax.lax.broadcasted_iota(jnp.int32, (BM, N), 1)
    d_ref[...] += jnp.sum(
        jnp.where(ci == ri + i * BM, a, 0.0), axis=0, keepdims=True
    )

    out_ref[pl.ds(i * BM, BM)] = jnp.dot(
        a.astype(jnp.bfloat16), s16_ref[...], preferred_element_type=jnp.float32
    )

    @pl.when(i == NB - 1)
    def _():
        rb = (
            jnp.dot(
                d_ref[...].astype(jnp.bfloat16),
                s16_ref[...],
                preferred_element_type=jnp.float32,
            )
            + b_ref[...].reshape(1, F)
        )
        out_ref[...] += rb


@jax.jit
def kernel(input, adj, W, b):
    s16 = pl.pallas_call(
        _prep_kernel,
        grid=(1,),
        in_specs=[
            pl.BlockSpec((N, F), lambda i: (0, 0)),
            pl.BlockSpec((F, F), lambda i: (0, 0)),
        ],
        out_specs=pl.BlockSpec((N, F), lambda i: (0, 0)),
        out_shape=jax.ShapeDtypeStruct((N, F), jnp.bfloat16),
    )(input, W)

    return pl.pallas_call(
        _main_kernel,
        grid=(NB,),
        in_specs=[
            pl.BlockSpec((BM, N), lambda i: (i, 0)),
            pl.BlockSpec((N, F), lambda i: (0, 0)),
            pl.BlockSpec((F,), lambda i: (0,)),
        ],
        out_specs=pl.BlockSpec((N, F), lambda i: (0, 0)),
        out_shape=jax.ShapeDtypeStruct((N, F), jnp.float32),
        scratch_shapes=[pltpu.VMEM((1, N), jnp.float32)],
    )(adj, s16, b)
